# Initial kernel scaffold; baseline (speedup 1.0000x reference)
import jax
import jax.numpy as jnp
from jax.experimental import pallas as pl


def kernel(x1, x2, edges, move_type, move_src, move_dst, move_armies, move_ids, g1_wl, g1_wr, g1_att, g1_b, g2_wl, g2_wr, g2_att, g2_b, g3_wl, g3_wr, g3_att, g3_b, lin_w, lin_b, lin2_w, lin2_b, aaa_w, aaa_b, bbb_w, bbb_b, ccc_w, ccc_b, ddd_w, ddd_b):
    raise NotImplementedError("write your pallas kernel here")


# speedup vs baseline: 18.3151x; 18.3151x over previous
import functools

import jax
import jax.numpy as jnp
from jax import lax
from jax.experimental import pallas as pl
from jax.experimental.pallas import tpu as pltpu
from jax.experimental.pallas import tpu_sc as plsc

_N = 50000
_NP = 50048
_E = 1600000
_T = 16384
_M = 4096
_NC = 2
_NS = 16
_NW = _NC * _NS
_EPW = _E // _NW
_CH = 128
_NCH = _EPW // _CH
_TAIL = _EPW - _NCH * _CH
_RPT = _NP // _NS
_ZR = 184
_RB = 3128
_NRB = _NP // _RB
_MB = 2048
_NMB = _T // _MB
_MPW = _T // _NW

_f32 = jnp.float32
_i32 = jnp.int32

_sc_mesh = plsc.VectorSubcoreMesh(core_axis_name="c", subcore_axis_name="s")


def _tables_first_body(x1_ref, wl_ref, wr_ref, cvec_ref, xlp_ref, xrp_ref):
    x = x1_ref[...]
    xlp_ref[...] = jnp.dot(x, wl_ref[...], preferred_element_type=_f32, precision=lax.Precision.HIGHEST) + cvec_ref[...]
    xrp_ref[...] = jnp.dot(x, wr_ref[...], preferred_element_type=_f32, precision=lax.Precision.HIGHEST)


def _tables_first(x1, wlp, wrp, cvec):
    return pl.pallas_call(
        _tables_first_body,
        grid=(_NRB,),
        in_specs=[
            pl.BlockSpec((_RB, 15), lambda i: (i, 0)),
            pl.BlockSpec((15, 16), lambda i: (0, 0)),
            pl.BlockSpec((15, 16), lambda i: (0, 0)),
            pl.BlockSpec((1, 16), lambda i: (0, 0)),
        ],
        out_specs=[pl.BlockSpec((_RB, 16), lambda i: (i, 0))] * 2,
        out_shape=[jax.ShapeDtypeStruct((_NP, 16), _f32)] * 2,
    )(x1, wlp, wrp, cvec)


def _tables_next_body(acc_ref, x1_ref, b_ref, wl1_ref, wl2_ref, wr1_ref,
                      wr2_ref, cvec_ref, xlp_ref, xrp_ref):
    a = acc_ref[0] + acc_ref[1]
    x = jnp.maximum(a[:, 0:10] / (a[:, 10:11] + 1e-16) + b_ref[...], 0.0)
    x1b = x1_ref[...]
    xlp_ref[...] = (jnp.dot(x, wl1_ref[...], preferred_element_type=_f32, precision=lax.Precision.HIGHEST)
                    + jnp.dot(x1b, wl2_ref[...], preferred_element_type=_f32, precision=lax.Precision.HIGHEST)
                    + cvec_ref[...])
    xrp_ref[...] = (jnp.dot(x, wr1_ref[...], preferred_element_type=_f32, precision=lax.Precision.HIGHEST)
                    + jnp.dot(x1b, wr2_ref[...], preferred_element_type=_f32, precision=lax.Precision.HIGHEST))


def _tables_next(acc, x1, bvec, wl1, wl2, wr1, wr2, cvec):
    return pl.pallas_call(
        _tables_next_body,
        grid=(_NRB,),
        in_specs=[
            pl.BlockSpec((2, _RB, 16), lambda i: (0, i, 0)),
            pl.BlockSpec((_RB, 15), lambda i: (i, 0)),
            pl.BlockSpec((1, 10), lambda i: (0, 0)),
            pl.BlockSpec((10, 16), lambda i: (0, 0)),
            pl.BlockSpec((15, 16), lambda i: (0, 0)),
            pl.BlockSpec((10, 16), lambda i: (0, 0)),
            pl.BlockSpec((15, 16), lambda i: (0, 0)),
            pl.BlockSpec((1, 16), lambda i: (0, 0)),
        ],
        out_specs=[pl.BlockSpec((_RB, 16), lambda i: (i, 0))] * 2,
        out_shape=[jax.ShapeDtypeStruct((_NP, 16), _f32)] * 2,
    )(acc, x1, bvec, wl1, wl2, wr1, wr2, cvec)


def _final_body(acc_ref, x1_ref, b_ref, s1_ref, s2_ref, lw1_ref, lw2_ref,
                lwx2_ref, x2_ref, linb_ref, l2w_ref, l2b_ref,
                mt_ref, vs_ref):
    i = pl.program_id(0)
    a = acc_ref[0] + acc_ref[1]
    x = jnp.maximum(a[:, 0:10] / (a[:, 10:11] + 1e-16) + b_ref[...], 0.0)
    x1b = x1_ref[...]
    mt_ref[...] = (jnp.dot(x, s1_ref[...], preferred_element_type=_f32, precision=lax.Precision.HIGHEST)
                   + jnp.dot(x1b, s2_ref[...], preferred_element_type=_f32, precision=lax.Precision.HIGHEST))
    cterm = jnp.dot(x2_ref[...], lwx2_ref[...], preferred_element_type=_f32, precision=lax.Precision.HIGHEST) + linb_ref[...]
    h = jnp.maximum(jnp.dot(x, lw1_ref[...], preferred_element_type=_f32, precision=lax.Precision.HIGHEST)
                    + jnp.dot(x1b, lw2_ref[...], preferred_element_type=_f32, precision=lax.Precision.HIGHEST)
                    + cterm, 0.0)
    v = jnp.dot(h, l2w_ref[...], preferred_element_type=_f32, precision=lax.Precision.HIGHEST) + l2b_ref[...]
    rows = i * _RB + lax.broadcasted_iota(_i32, (_RB, 1), 0)
    v = jnp.where(rows < _N, v, 0.0)

    @pl.when(i == 0)
    def _():
        vs_ref[...] = jnp.zeros_like(vs_ref)

    vs_ref[...] += jnp.sum(v, keepdims=True).reshape(1, 1)


def _final(acc, x1, bvec, s1, s2, lw1, lw2, lwx2, x2r, linb, l2w, l2b):
    return pl.pallas_call(
        _final_body,
        grid=(_NRB,),
        in_specs=[
            pl.BlockSpec((2, _RB, 16), lambda i: (0, i, 0)),
            pl.BlockSpec((_RB, 15), lambda i: (i, 0)),
            pl.BlockSpec((1, 10), lambda i: (0, 0)),
            pl.BlockSpec((10, 32), lambda i: (0, 0)),
            pl.BlockSpec((15, 32), lambda i: (0, 0)),
            pl.BlockSpec((10, 15), lambda i: (0, 0)),
            pl.BlockSpec((15, 15), lambda i: (0, 0)),
            pl.BlockSpec((4, 15), lambda i: (0, 0)),
            pl.BlockSpec((1, 4), lambda i: (0, 0)),
            pl.BlockSpec((1, 15), lambda i: (0, 0)),
            pl.BlockSpec((15, 1), lambda i: (0, 0)),
            pl.BlockSpec((1, 1), lambda i: (0, 0)),
        ],
        out_specs=[
            pl.BlockSpec((_RB, 32), lambda i: (i, 0)),
            pl.BlockSpec((1, 1), lambda i: (0, 0)),
        ],
        out_shape=[
            jax.ShapeDtypeStruct((_NP, 32), _f32),
            jax.ShapeDtypeStruct((1, 1), _f32),
        ],
    )(acc, x1, bvec, s1, s2, lw1, lw2, lwx2, x2r, linb, l2w, l2b)


def _move_body(s_ref, d_ref, arm_ref, mt_ref, awxs, awxd, awx1s, awx1d,
               awarm, awext, ab, bw, bb, cwxd, cwx1d, cwarm, cb, dw, db,
               out_ref):
    s = s_ref[...]
    d = d_ref[...]
    arm = arm_ref[...]
    extra = 0.6 * arm - 0.7 * (d[:, 13:14] + d[:, 14:15])
    ha = jnp.maximum(
        jnp.dot(s[:, 0:10], awxs[...], preferred_element_type=_f32, precision=lax.Precision.HIGHEST)
        + jnp.dot(d[:, 0:10], awxd[...], preferred_element_type=_f32, precision=lax.Precision.HIGHEST)
        + jnp.dot(s[:, 13:25], awx1s[...], preferred_element_type=_f32, precision=lax.Precision.HIGHEST)
        + jnp.dot(d[:, 11:25], awx1d[...], preferred_element_type=_f32, precision=lax.Precision.HIGHEST)
        + arm * awarm[...] + extra * awext[...] + ab[...], 0.0)
    sa = jnp.dot(ha, bw[...], preferred_element_type=_f32, precision=lax.Precision.HIGHEST) + bb[...]
    hd = jnp.maximum(
        jnp.dot(d[:, 0:10], cwxd[...], preferred_element_type=_f32, precision=lax.Precision.HIGHEST)
        + jnp.dot(d[:, 13:25], cwx1d[...], preferred_element_type=_f32, precision=lax.Precision.HIGHEST)
        + arm * cwarm[...] + cb[...], 0.0)
    sd = jnp.dot(hd, dw[...], preferred_element_type=_f32, precision=lax.Precision.HIGHEST) + db[...]
    out_ref[...] = jnp.where(mt_ref[...] == 0, sa, sd)


def _move(srows, drows, arm, mtype, wpieces):
    const = lambda shape: pl.BlockSpec(shape, lambda i: (0, 0))
    return pl.pallas_call(
        _move_body,
        grid=(_NMB,),
        in_specs=[
            pl.BlockSpec((_MB, 32), lambda i: (i, 0)),
            pl.BlockSpec((_MB, 32), lambda i: (i, 0)),
            pl.BlockSpec((_MB, 1), lambda i: (i, 0)),
            pl.BlockSpec((_MB, 1), lambda i: (i, 0)),
            const((10, 20)), const((10, 20)), const((12, 20)), const((14, 20)),
            const((1, 20)), const((1, 20)), const((1, 20)),
            const((20, 1)), const((1, 1)),
            const((10, 20)), const((12, 20)), const((1, 20)), const((1, 20)),
            const((20, 1)), const((1, 1)),
        ],
        out_specs=pl.BlockSpec((_MB, 1), lambda i: (i, 0)),
        out_shape=jax.ShapeDtypeStruct((_T, 1), _f32),
    )(srows, drows, arm, mtype, *wpieces)


def _fin2_body(pp_ref, vs_ref, logp_ref, v_ref):
    a = jnp.sum(pp_ref[...], axis=0)
    p = jnp.sum(a, axis=0, keepdims=True)
    m = jnp.max(p)
    lse = m + jnp.log(jnp.sum(jnp.exp(p - m)))
    logp_ref[...] = p - lse
    v_ref[...] = jnp.tanh(vs_ref[...] / _N)


def _fin2(pparts, vsum):
    return pl.pallas_call(
        _fin2_body,
        grid=(1,),
        in_specs=[
            pl.BlockSpec((_NC, _NS, _M), lambda i: (0, 0, 0)),
            pl.BlockSpec((1, 1), lambda i: (0, 0)),
        ],
        out_specs=[
            pl.BlockSpec((1, _M), lambda i: (0, 0)),
            pl.BlockSpec((1, 1), lambda i: (0, 0)),
        ],
        out_shape=[
            jax.ShapeDtypeStruct((1, _M), _f32),
            jax.ShapeDtypeStruct((1, 1), _f32),
        ],
    )(pparts.reshape(_NC, _NS, _M), vsum)


def _edge_body(xlp, xrp, src, dst, attp, out, acc, zbuf, attv,
               idxs, idxd, lrows, rrows, orows,
               idxs_t, idxd_t, lrows_t, rrows_t, orows_t,
               sbuf, exb, sem):
    cid = lax.axis_index("c")
    sid = lax.axis_index("s")
    wid = cid * _NS + sid

    def _zb(j, c):
        zbuf[j, :] = jnp.zeros((16,), _f32)
        return c
    lax.fori_loop(0, _ZR, _zb, 0)
    pltpu.sync_copy(attp, attv)

    def _zc(k, c):
        pltpu.sync_copy(zbuf, acc.at[pl.ds(sid * _RPT + k * _ZR, _ZR)])
        return c
    lax.fori_loop(0, _RPT // _ZR, _zc, 0)
    plsc.subcore_barrier()

    base = wid * _EPW
    attvv = attv[...]
    ii = lax.iota(_i32, 16) * 16

    def _process(klen, p_idxs, p_idxd, p_lrows, p_rrows, p_orows, off):
        pltpu.sync_copy(src.at[pl.ds(off, klen)], p_idxs)
        pltpu.sync_copy(dst.at[pl.ds(off, klen)], p_idxd)
        pltpu.async_copy(xlp.at[p_idxs], p_lrows, sem).wait()
        pltpu.async_copy(xrp.at[p_idxd], p_rrows, sem).wait()

        def _group(g, c):
            rb = g * 16
            for j in range(16):
                t = p_lrows[rb + j, :] + p_rrows[rb + j, :]
                sbuf[pl.ds(j * 16, 16)] = jnp.maximum(t, 0.2 * t) * attvv
            ev = plsc.load_gather(sbuf, [ii])
            for dcol in range(1, 10):
                ev = ev + plsc.load_gather(sbuf, [ii + dcol])
            exb[...] = jnp.exp(ev)
            for j in range(16):
                exj = plsc.load_gather(exb, [jnp.full((16,), j, _i32)])
                p_orows[rb + j, :] = exj * p_lrows[rb + j, :]
            return c
        lax.fori_loop(0, klen // 16, _group, 0)
        pltpu.sync_copy(p_orows, acc.at[p_idxd], add=True)

    def _chunk(c, carry):
        _process(_CH, idxs, idxd, lrows, rrows, orows, base + c * _CH)
        return carry
    lax.fori_loop(0, _NCH, _chunk, 0)
    _process(_TAIL, idxs_t, idxd_t, lrows_t, rrows_t, orows_t,
             base + _NCH * _CH)

    plsc.subcore_barrier()
    pltpu.sync_copy(acc.at[pl.ds(sid * _RPT, _RPT)],
                    out.at[cid, pl.ds(sid * _RPT, _RPT)])


def _edge_pass(xlp, xrp, src, dst, attp):
    kern = pl.kernel(
        _edge_body,
        out_type=jax.ShapeDtypeStruct((_NC, _NP, 16), _f32),
        mesh=_sc_mesh,
        compiler_params=pltpu.CompilerParams(needs_layout_passes=False, use_tc_tiling_on_sc=False),
        scratch_types=[
            pltpu.VMEM_SHARED((_NP, 16), _f32),
            pltpu.VMEM((_ZR, 16), _f32),
            pltpu.VMEM((16,), _f32),
            pltpu.VMEM((_CH,), _i32), pltpu.VMEM((_CH,), _i32),
            pltpu.VMEM((_CH, 16), _f32), pltpu.VMEM((_CH, 16), _f32),
            pltpu.VMEM((_CH, 16), _f32),
            pltpu.VMEM((_TAIL,), _i32), pltpu.VMEM((_TAIL,), _i32),
            pltpu.VMEM((_TAIL, 16), _f32), pltpu.VMEM((_TAIL, 16), _f32),
            pltpu.VMEM((_TAIL, 16), _f32),
            pltpu.VMEM((256,), _f32),
            pltpu.VMEM((16,), _f32),
            pltpu.SemaphoreType.DMA,
        ],
    )
    return kern(xlp, xrp, src, dst, attp)


def _mgather_body(mtable, msrc, mdst, out, idx, rows, sem):
    cid = lax.axis_index("c")
    sid = lax.axis_index("s")
    base = (cid * _NS + sid) * _MPW
    for k in range(_MPW // 128):
        off = base + k * 128
        pltpu.sync_copy(msrc.at[pl.ds(off, 128)], idx)
        pltpu.async_copy(mtable.at[idx], rows, sem).wait()
        pltpu.sync_copy(rows, out.at[0, pl.ds(off, 128)])
        pltpu.sync_copy(mdst.at[pl.ds(off, 128)], idx)
        pltpu.async_copy(mtable.at[idx], rows, sem).wait()
        pltpu.sync_copy(rows, out.at[1, pl.ds(off, 128)])


def _mgather(mtable, msrc, mdst):
    kern = pl.kernel(
        _mgather_body,
        out_type=jax.ShapeDtypeStruct((2, _T, 32), _f32),
        mesh=_sc_mesh,
        compiler_params=pltpu.CompilerParams(needs_layout_passes=False, use_tc_tiling_on_sc=False),
        scratch_types=[
            pltpu.VMEM((128,), _i32),
            pltpu.VMEM((128, 32), _f32),
            pltpu.SemaphoreType.DMA,
        ],
    )
    return kern(mtable, msrc, mdst)


def _pscatter_body(per, mids, out, pbuf, ids, vals):
    cid = lax.axis_index("c")
    sid = lax.axis_index("s")
    base = (cid * _NS + sid) * _MPW

    def _z(j, c):
        pbuf[pl.ds(j * 16, 16)] = jnp.zeros((16,), _f32)
        return c
    lax.fori_loop(0, _M // 16, _z, 0)
    pltpu.sync_copy(mids.at[pl.ds(base, _MPW)], ids)
    pltpu.sync_copy(per.at[pl.ds(base, _MPW)], vals)

    def _g(j, c):
        plsc.addupdate_scatter(pbuf, [ids[pl.ds(j * 16, 16)]],
                               vals[pl.ds(j * 16, 16)])
        return c
    lax.fori_loop(0, _MPW // 16, _g, 0)
    pltpu.sync_copy(pbuf, out.at[cid, sid, 0])


def _pscatter(per, mids):
    kern = pl.kernel(
        _pscatter_body,
        out_type=jax.ShapeDtypeStruct((_NC, _NS, 1, _M), _f32),
        mesh=_sc_mesh,
        compiler_params=pltpu.CompilerParams(needs_layout_passes=False, use_tc_tiling_on_sc=False),
        scratch_types=[
            pltpu.VMEM((_M,), _f32),
            pltpu.VMEM((_MPW,), _i32),
            pltpu.VMEM((_MPW,), _f32),
        ],
    )
    return kern(per, mids)


def kernel(x1, x2, edges, move_type, move_src, move_dst, move_armies,
           move_ids, g1_wl, g1_wr, g1_att, g1_b, g2_wl, g2_wr, g2_att, g2_b,
           g3_wl, g3_wr, g3_att, g3_b, lin_w, lin_b, lin2_w, lin2_b,
           aaa_w, aaa_b, bbb_w, bbb_b, ccc_w, ccc_b, ddd_w, ddd_b):
    src = edges[0]
    dst = edges[1]
    x1 = jnp.pad(x1, ((0, _NP - _N), (0, 0)))
    cvec = jnp.zeros((1, 16), _f32).at[0, 10].set(1.0)
    pad6 = lambda w: jnp.pad(w, ((0, 0), (0, 6)))

    xlp, xrp = _tables_first(x1, pad6(g1_wl), pad6(g1_wr), cvec)
    acc = _edge_pass(xlp, xrp, src, dst, jnp.pad(g1_att, (0, 6)))
    for wl, wr, att, bprev in ((g2_wl, g2_wr, g2_att, g1_b),
                               (g3_wl, g3_wr, g3_att, g2_b)):
        xlp, xrp = _tables_next(acc, x1, bprev.reshape(1, 10),
                                pad6(wl[0:10]), pad6(wl[10:25]),
                                pad6(wr[0:10]), pad6(wr[10:25]), cvec)
        acc = _edge_pass(xlp, xrp, src, dst, jnp.pad(att, (0, 6)))

    s1 = jnp.pad(jnp.eye(10, dtype=_f32), ((0, 0), (0, 22)))
    s2 = jnp.pad(jnp.eye(15, dtype=_f32), ((0, 0), (10, 7)))
    mtable, vsum = _final(acc, x1, g3_b.reshape(1, 10), s1, s2,
                          lin_w[0:10], lin_w[10:25], lin_w[25:29],
                          x2.reshape(1, 4), lin_b.reshape(1, 15),
                          lin2_w, lin2_b.reshape(1, 1))

    sdrows = _mgather(mtable, move_src, move_dst)
    wpieces = (aaa_w[0:10], aaa_w[10:20], aaa_w[20:32], aaa_w[32:46],
               aaa_w[46:47], aaa_w[47:48], aaa_b.reshape(1, 20),
               bbb_w, bbb_b.reshape(1, 1),
               ccc_w[0:10], ccc_w[10:22], ccc_w[22:23], ccc_b.reshape(1, 20),
               ddd_w, ddd_b.reshape(1, 1))
    per = _move(sdrows[0], sdrows[1], move_armies.reshape(_T, 1),
                move_type.reshape(_T, 1), wpieces)

    pparts = _pscatter(per.reshape(_T), move_ids)
    logp, v = _fin2(pparts, vsum)
    return (v.reshape(()), logp.reshape(_M))


# speedup vs baseline: 33.9383x; 1.8530x over previous
import functools

import jax
import jax.numpy as jnp
from jax import lax
from jax.experimental import pallas as pl
from jax.experimental.pallas import tpu as pltpu
from jax.experimental.pallas import tpu_sc as plsc

_N = 50000
_NP = 50048
_E = 1600000
_T = 16384
_M = 4096
_NC = 2
_NS = 16
_NW = _NC * _NS
_EPW = _E // _NW
_CH = 128
_NCH = _EPW // _CH
_TAIL = _EPW - _NCH * _CH
_RPT = _NP // _NS
_ZR = 184
_RB = 3128
_NRB = _NP // _RB
_MB = 2048
_NMB = _T // _MB
_MPW = _T // _NW

_f32 = jnp.float32
_i32 = jnp.int32

_sc_mesh = plsc.VectorSubcoreMesh(core_axis_name="c", subcore_axis_name="s")


def _tables_first_body(x1_ref, wl_ref, wr_ref, cvec_ref, xlp_ref, xrp_ref):
    x = x1_ref[...]
    xlp_ref[...] = jnp.dot(x, wl_ref[...], preferred_element_type=_f32, precision=lax.Precision.HIGHEST) + cvec_ref[...]
    xrp_ref[...] = jnp.dot(x, wr_ref[...], preferred_element_type=_f32, precision=lax.Precision.HIGHEST)


def _tables_first(x1, wlp, wrp, cvec):
    return pl.pallas_call(
        _tables_first_body,
        grid=(_NRB,),
        in_specs=[
            pl.BlockSpec((_RB, 15), lambda i: (i, 0)),
            pl.BlockSpec((15, 16), lambda i: (0, 0)),
            pl.BlockSpec((15, 16), lambda i: (0, 0)),
            pl.BlockSpec((1, 16), lambda i: (0, 0)),
        ],
        out_specs=[pl.BlockSpec((_RB, 16), lambda i: (i, 0))] * 2,
        out_shape=[jax.ShapeDtypeStruct((_NP, 16), _f32)] * 2,
    )(x1, wlp, wrp, cvec)


def _tables_next_body(acc_ref, x1_ref, b_ref, wl1_ref, wl2_ref, wr1_ref,
                      wr2_ref, cvec_ref, xlp_ref, xrp_ref):
    a = acc_ref[0] + acc_ref[1]
    x = jnp.maximum(a[:, 0:10] / (a[:, 10:11] + 1e-16) + b_ref[...], 0.0)
    x1b = x1_ref[...]
    xlp_ref[...] = (jnp.dot(x, wl1_ref[...], preferred_element_type=_f32, precision=lax.Precision.HIGHEST)
                    + jnp.dot(x1b, wl2_ref[...], preferred_element_type=_f32, precision=lax.Precision.HIGHEST)
                    + cvec_ref[...])
    xrp_ref[...] = (jnp.dot(x, wr1_ref[...], preferred_element_type=_f32, precision=lax.Precision.HIGHEST)
                    + jnp.dot(x1b, wr2_ref[...], preferred_element_type=_f32, precision=lax.Precision.HIGHEST))


def _tables_next(acc, x1, bvec, wl1, wl2, wr1, wr2, cvec):
    return pl.pallas_call(
        _tables_next_body,
        grid=(_NRB,),
        in_specs=[
            pl.BlockSpec((2, _RB, 16), lambda i: (0, i, 0)),
            pl.BlockSpec((_RB, 15), lambda i: (i, 0)),
            pl.BlockSpec((1, 10), lambda i: (0, 0)),
            pl.BlockSpec((10, 16), lambda i: (0, 0)),
            pl.BlockSpec((15, 16), lambda i: (0, 0)),
            pl.BlockSpec((10, 16), lambda i: (0, 0)),
            pl.BlockSpec((15, 16), lambda i: (0, 0)),
            pl.BlockSpec((1, 16), lambda i: (0, 0)),
        ],
        out_specs=[pl.BlockSpec((_RB, 16), lambda i: (i, 0))] * 2,
        out_shape=[jax.ShapeDtypeStruct((_NP, 16), _f32)] * 2,
    )(acc, x1, bvec, wl1, wl2, wr1, wr2, cvec)


def _final_body(acc_ref, x1_ref, b_ref, s1_ref, s2_ref, lw1_ref, lw2_ref,
                lwx2_ref, x2_ref, linb_ref, l2w_ref, l2b_ref,
                mt_ref, vs_ref):
    i = pl.program_id(0)
    a = acc_ref[0] + acc_ref[1]
    x = jnp.maximum(a[:, 0:10] / (a[:, 10:11] + 1e-16) + b_ref[...], 0.0)
    x1b = x1_ref[...]
    mt_ref[...] = (jnp.dot(x, s1_ref[...], preferred_element_type=_f32, precision=lax.Precision.HIGHEST)
                   + jnp.dot(x1b, s2_ref[...], preferred_element_type=_f32, precision=lax.Precision.HIGHEST))
    cterm = jnp.dot(x2_ref[...], lwx2_ref[...], preferred_element_type=_f32, precision=lax.Precision.HIGHEST) + linb_ref[...]
    h = jnp.maximum(jnp.dot(x, lw1_ref[...], preferred_element_type=_f32, precision=lax.Precision.HIGHEST)
                    + jnp.dot(x1b, lw2_ref[...], preferred_element_type=_f32, precision=lax.Precision.HIGHEST)
                    + cterm, 0.0)
    v = jnp.dot(h, l2w_ref[...], preferred_element_type=_f32, precision=lax.Precision.HIGHEST) + l2b_ref[...]
    rows = i * _RB + lax.broadcasted_iota(_i32, (_RB, 1), 0)
    v = jnp.where(rows < _N, v, 0.0)

    @pl.when(i == 0)
    def _():
        vs_ref[...] = jnp.zeros_like(vs_ref)

    vs_ref[...] += jnp.sum(v, keepdims=True).reshape(1, 1)


def _final(acc, x1, bvec, s1, s2, lw1, lw2, lwx2, x2r, linb, l2w, l2b):
    return pl.pallas_call(
        _final_body,
        grid=(_NRB,),
        in_specs=[
            pl.BlockSpec((2, _RB, 16), lambda i: (0, i, 0)),
            pl.BlockSpec((_RB, 15), lambda i: (i, 0)),
            pl.BlockSpec((1, 10), lambda i: (0, 0)),
            pl.BlockSpec((10, 32), lambda i: (0, 0)),
            pl.BlockSpec((15, 32), lambda i: (0, 0)),
            pl.BlockSpec((10, 15), lambda i: (0, 0)),
            pl.BlockSpec((15, 15), lambda i: (0, 0)),
            pl.BlockSpec((4, 15), lambda i: (0, 0)),
            pl.BlockSpec((1, 4), lambda i: (0, 0)),
            pl.BlockSpec((1, 15), lambda i: (0, 0)),
            pl.BlockSpec((15, 1), lambda i: (0, 0)),
            pl.BlockSpec((1, 1), lambda i: (0, 0)),
        ],
        out_specs=[
            pl.BlockSpec((_RB, 32), lambda i: (i, 0)),
            pl.BlockSpec((1, 1), lambda i: (0, 0)),
        ],
        out_shape=[
            jax.ShapeDtypeStruct((_NP, 32), _f32),
            jax.ShapeDtypeStruct((1, 1), _f32),
        ],
    )(acc, x1, bvec, s1, s2, lw1, lw2, lwx2, x2r, linb, l2w, l2b)


def _move_body(s_ref, d_ref, arm_ref, mt_ref, awxs, awxd, awx1s, awx1d,
               awarm, awext, ab, bw, bb, cwxd, cwx1d, cwarm, cb, dw, db,
               out_ref):
    s = s_ref[...]
    d = d_ref[...]
    arm = arm_ref[...]
    extra = 0.6 * arm - 0.7 * (d[:, 13:14] + d[:, 14:15])
    ha = jnp.maximum(
        jnp.dot(s[:, 0:10], awxs[...], preferred_element_type=_f32, precision=lax.Precision.HIGHEST)
        + jnp.dot(d[:, 0:10], awxd[...], preferred_element_type=_f32, precision=lax.Precision.HIGHEST)
        + jnp.dot(s[:, 13:25], awx1s[...], preferred_element_type=_f32, precision=lax.Precision.HIGHEST)
        + jnp.dot(d[:, 11:25], awx1d[...], preferred_element_type=_f32, precision=lax.Precision.HIGHEST)
        + arm * awarm[...] + extra * awext[...] + ab[...], 0.0)
    sa = jnp.dot(ha, bw[...], preferred_element_type=_f32, precision=lax.Precision.HIGHEST) + bb[...]
    hd = jnp.maximum(
        jnp.dot(d[:, 0:10], cwxd[...], preferred_element_type=_f32, precision=lax.Precision.HIGHEST)
        + jnp.dot(d[:, 13:25], cwx1d[...], preferred_element_type=_f32, precision=lax.Precision.HIGHEST)
        + arm * cwarm[...] + cb[...], 0.0)
    sd = jnp.dot(hd, dw[...], preferred_element_type=_f32, precision=lax.Precision.HIGHEST) + db[...]
    out_ref[...] = jnp.where(mt_ref[...] == 0, sa, sd)


def _move(srows, drows, arm, mtype, wpieces):
    const = lambda shape: pl.BlockSpec(shape, lambda i: (0, 0))
    return pl.pallas_call(
        _move_body,
        grid=(_NMB,),
        in_specs=[
            pl.BlockSpec((_MB, 32), lambda i: (i, 0)),
            pl.BlockSpec((_MB, 32), lambda i: (i, 0)),
            pl.BlockSpec((_MB, 1), lambda i: (i, 0)),
            pl.BlockSpec((_MB, 1), lambda i: (i, 0)),
            const((10, 20)), const((10, 20)), const((12, 20)), const((14, 20)),
            const((1, 20)), const((1, 20)), const((1, 20)),
            const((20, 1)), const((1, 1)),
            const((10, 20)), const((12, 20)), const((1, 20)), const((1, 20)),
            const((20, 1)), const((1, 1)),
        ],
        out_specs=pl.BlockSpec((_MB, 1), lambda i: (i, 0)),
        out_shape=jax.ShapeDtypeStruct((_T, 1), _f32),
    )(srows, drows, arm, mtype, *wpieces)


def _fin2_body(pp_ref, vs_ref, logp_ref, v_ref):
    a = jnp.sum(pp_ref[...], axis=0)
    p = jnp.sum(a, axis=0, keepdims=True)
    m = jnp.max(p)
    lse = m + jnp.log(jnp.sum(jnp.exp(p - m)))
    logp_ref[...] = p - lse
    v_ref[...] = jnp.tanh(vs_ref[...] / _N)


def _fin2(pparts, vsum):
    return pl.pallas_call(
        _fin2_body,
        grid=(1,),
        in_specs=[
            pl.BlockSpec((_NC, _NS, _M), lambda i: (0, 0, 0)),
            pl.BlockSpec((1, 1), lambda i: (0, 0)),
        ],
        out_specs=[
            pl.BlockSpec((1, _M), lambda i: (0, 0)),
            pl.BlockSpec((1, 1), lambda i: (0, 0)),
        ],
        out_shape=[
            jax.ShapeDtypeStruct((1, _M), _f32),
            jax.ShapeDtypeStruct((1, 1), _f32),
        ],
    )(pparts.reshape(_NC, _NS, _M), vsum)


def _edge_body(xlp, xrp, src, dst, attp, out, acc, zbuf, attv,
               idxs0, idxs1, idxs2, idxd0, idxd1, idxd2,
               sidx0, sidx1, sidx2,
               lr0, lr1, lr2, rr0, rr1, rr2, or0, or1, or2,
               idxs_t, idxd_t, lrows_t, rrows_t, orows_t,
               sbuf, exb,
               isem0, isem1, isem2, gsem0, gsem1, gsem2,
               ssem0, ssem1, ssem2, zsem, tsem):
    cid = lax.axis_index("c")
    sid = lax.axis_index("s")
    wid = cid * _NS + sid
    IDXS = (idxs0, idxs1, idxs2)
    IDXD = (idxd0, idxd1, idxd2)
    SIDX = (sidx0, sidx1, sidx2)
    LR = (lr0, lr1, lr2)
    RR = (rr0, rr1, rr2)
    OR = (or0, or1, or2)
    ISEM = (isem0, isem1, isem2)
    GSEM = (gsem0, gsem1, gsem2)
    SSEM = (ssem0, ssem1, ssem2)

    def _zb(j, c):
        zbuf[j, :] = jnp.zeros((16,), _f32)
        return c
    lax.fori_loop(0, _ZR, _zb, 0)
    pltpu.sync_copy(attp, attv)
    nz = _RPT // _ZR
    for k in range(nz):
        pltpu.make_async_copy(
            zbuf, acc.at[pl.ds(sid * _RPT + k * _ZR, _ZR)], zsem).start()
    for k in range(nz):
        pltpu.make_async_copy(
            zbuf, acc.at[pl.ds(sid * _RPT + k * _ZR, _ZR)], zsem).wait()
    plsc.subcore_barrier()

    base = wid * _EPW
    attvv = attv[...]
    ii = lax.iota(_i32, 16) * 16

    def _idx_descs(b, c):
        off = base + c * _CH
        return (pltpu.make_async_copy(src.at[pl.ds(off, _CH)], IDXS[b], ISEM[b]),
                pltpu.make_async_copy(dst.at[pl.ds(off, _CH)], IDXD[b], ISEM[b]))

    def _gath_descs(b):
        return (pltpu.make_async_copy(xlp.at[IDXS[b]], LR[b], GSEM[b]),
                pltpu.make_async_copy(xrp.at[IDXD[b]], RR[b], GSEM[b]))

    def _compute(lrows, rrows, orows):
        def _group(g, c):
            rb = g * 16
            for j in range(16):
                t = lrows[rb + j, :] + rrows[rb + j, :]
                sbuf[pl.ds(j * 16, 16)] = jnp.maximum(t, 0.2 * t) * attvv
            ev = plsc.load_gather(sbuf, [ii])
            for dcol in range(1, 10):
                ev = ev + plsc.load_gather(sbuf, [ii + dcol])
            exb[...] = jnp.exp(ev)
            for j in range(16):
                exj = plsc.load_gather(exb, [jnp.full((16,), j, _i32)])
                orows[rb + j, :] = exj * lrows[rb + j, :]
            return c
        lax.fori_loop(0, _CH // 16, _group, 0)

    def _do_chunk(k, it):
        c = it * 3 + k
        b = k
        g1, g2 = _gath_descs(b)
        g1.wait()
        g2.wait()

        @pl.when(c >= 3)
        def _():
            pltpu.make_async_copy(OR[b], acc.at[SIDX[b]], SSEM[b]).wait()
        for q in range(_CH // 16):
            SIDX[b][pl.ds(q * 16, 16)] = IDXD[b][pl.ds(q * 16, 16)]

        @pl.when(c + 3 < _NCH)
        def _():
            d1, d2 = _idx_descs(b, c + 3)
            d1.start()
            d2.start()

        @pl.when(c + 2 < _NCH)
        def _():
            b2 = (k + 2) % 3
            i1, i2 = _idx_descs(b2, c + 2)
            i1.wait()
            i2.wait()
            n1, n2 = _gath_descs(b2)
            n1.start()
            n2.start()

        _compute(LR[b], RR[b], OR[b])
        pltpu.async_copy(OR[b], acc.at[SIDX[b]], SSEM[b], add=True)

    for c0 in range(3):
        d1, d2 = _idx_descs(c0, c0)
        d1.start()
        d2.start()
    for c0 in range(2):
        d1, d2 = _idx_descs(c0, c0)
        d1.wait()
        d2.wait()
        g1, g2 = _gath_descs(c0)
        g1.start()
        g2.start()

    def _it_body(it, carry):
        _do_chunk(0, it)
        _do_chunk(1, it)
        _do_chunk(2, it)
        return carry
    lax.fori_loop(0, _NCH // 3, _it_body, 0)
    for b in range(3):
        pltpu.make_async_copy(OR[b], acc.at[SIDX[b]], SSEM[b]).wait()

    off = base + _NCH * _CH
    pltpu.sync_copy(src.at[pl.ds(off, _TAIL)], idxs_t)
    pltpu.sync_copy(dst.at[pl.ds(off, _TAIL)], idxd_t)
    pltpu.async_copy(xlp.at[idxs_t], lrows_t, tsem).wait()
    pltpu.async_copy(xrp.at[idxd_t], rrows_t, tsem).wait()

    def _tgroup(g, c):
        rb = g * 16
        for j in range(16):
            t = lrows_t[rb + j, :] + rrows_t[rb + j, :]
            sbuf[pl.ds(j * 16, 16)] = jnp.maximum(t, 0.2 * t) * attvv
        ev = plsc.load_gather(sbuf, [ii])
        for dcol in range(1, 10):
            ev = ev + plsc.load_gather(sbuf, [ii + dcol])
        exb[...] = jnp.exp(ev)
        for j in range(16):
            exj = plsc.load_gather(exb, [jnp.full((16,), j, _i32)])
            orows_t[rb + j, :] = exj * lrows_t[rb + j, :]
        return c
    lax.fori_loop(0, _TAIL // 16, _tgroup, 0)
    pltpu.sync_copy(orows_t, acc.at[idxd_t], add=True)

    plsc.subcore_barrier()
    pltpu.sync_copy(acc.at[pl.ds(sid * _RPT, _RPT)],
                    out.at[cid, pl.ds(sid * _RPT, _RPT)])


def _edge_pass(xlp, xrp, src, dst, attp):
    kern = pl.kernel(
        _edge_body,
        out_type=jax.ShapeDtypeStruct((_NC, _NP, 16), _f32),
        mesh=_sc_mesh,
        compiler_params=pltpu.CompilerParams(needs_layout_passes=False, use_tc_tiling_on_sc=False),
        scratch_types=[
            pltpu.VMEM_SHARED((_NP, 16), _f32),
            pltpu.VMEM((_ZR, 16), _f32),
            pltpu.VMEM((16,), _f32),
        ] + [pltpu.VMEM((_CH,), _i32)] * 9
          + [pltpu.VMEM((_CH, 16), _f32)] * 9
          + [pltpu.VMEM((_TAIL,), _i32)] * 2
          + [pltpu.VMEM((_TAIL, 16), _f32)] * 3
          + [pltpu.VMEM((256,), _f32), pltpu.VMEM((16,), _f32)]
          + [pltpu.SemaphoreType.DMA] * 11,
    )
    return kern(xlp, xrp, src, dst, attp)


def _mgather_body(mtable, msrc, mdst, out, idx, rows, sem):
    cid = lax.axis_index("c")
    sid = lax.axis_index("s")
    base = (cid * _NS + sid) * _MPW
    for k in range(_MPW // 128):
        off = base + k * 128
        pltpu.sync_copy(msrc.at[pl.ds(off, 128)], idx)
        pltpu.async_copy(mtable.at[idx], rows, sem).wait()
        pltpu.sync_copy(rows, out.at[0, pl.ds(off, 128)])
        pltpu.sync_copy(mdst.at[pl.ds(off, 128)], idx)
        pltpu.async_copy(mtable.at[idx], rows, sem).wait()
        pltpu.sync_copy(rows, out.at[1, pl.ds(off, 128)])


def _mgather(mtable, msrc, mdst):
    kern = pl.kernel(
        _mgather_body,
        out_type=jax.ShapeDtypeStruct((2, _T, 32), _f32),
        mesh=_sc_mesh,
        compiler_params=pltpu.CompilerParams(needs_layout_passes=False, use_tc_tiling_on_sc=False),
        scratch_types=[
            pltpu.VMEM((128,), _i32),
            pltpu.VMEM((128, 32), _f32),
            pltpu.SemaphoreType.DMA,
        ],
    )
    return kern(mtable, msrc, mdst)


def _pscatter_body(per, mids, out, pbuf, ids, vals):
    cid = lax.axis_index("c")
    sid = lax.axis_index("s")
    base = (cid * _NS + sid) * _MPW

    def _z(j, c):
        pbuf[pl.ds(j * 16, 16)] = jnp.zeros((16,), _f32)
        return c
    lax.fori_loop(0, _M // 16, _z, 0)
    pltpu.sync_copy(mids.at[pl.ds(base, _MPW)], ids)
    pltpu.sync_copy(per.at[pl.ds(base, _MPW)], vals)

    def _g(j, c):
        plsc.addupdate_scatter(pbuf, [ids[pl.ds(j * 16, 16)]],
                               vals[pl.ds(j * 16, 16)])
        return c
    lax.fori_loop(0, _MPW // 16, _g, 0)
    pltpu.sync_copy(pbuf, out.at[cid, sid, 0])


def _pscatter(per, mids):
    kern = pl.kernel(
        _pscatter_body,
        out_type=jax.ShapeDtypeStruct((_NC, _NS, 1, _M), _f32),
        mesh=_sc_mesh,
        compiler_params=pltpu.CompilerParams(needs_layout_passes=False, use_tc_tiling_on_sc=False),
        scratch_types=[
            pltpu.VMEM((_M,), _f32),
            pltpu.VMEM((_MPW,), _i32),
            pltpu.VMEM((_MPW,), _f32),
        ],
    )
    return kern(per, mids)


def kernel(x1, x2, edges, move_type, move_src, move_dst, move_armies,
           move_ids, g1_wl, g1_wr, g1_att, g1_b, g2_wl, g2_wr, g2_att, g2_b,
           g3_wl, g3_wr, g3_att, g3_b, lin_w, lin_b, lin2_w, lin2_b,
           aaa_w, aaa_b, bbb_w, bbb_b, ccc_w, ccc_b, ddd_w, ddd_b):
    src = edges[0]
    dst = edges[1]
    x1 = jnp.pad(x1, ((0, _NP - _N), (0, 0)))
    cvec = jnp.zeros((1, 16), _f32).at[0, 10].set(1.0)
    pad6 = lambda w: jnp.pad(w, ((0, 0), (0, 6)))

    xlp, xrp = _tables_first(x1, pad6(g1_wl), pad6(g1_wr), cvec)
    acc = _edge_pass(xlp, xrp, src, dst, jnp.pad(g1_att, (0, 6)))
    for wl, wr, att, bprev in ((g2_wl, g2_wr, g2_att, g1_b),
                               (g3_wl, g3_wr, g3_att, g2_b)):
        xlp, xrp = _tables_next(acc, x1, bprev.reshape(1, 10),
                                pad6(wl[0:10]), pad6(wl[10:25]),
                                pad6(wr[0:10]), pad6(wr[10:25]), cvec)
        acc = _edge_pass(xlp, xrp, src, dst, jnp.pad(att, (0, 6)))

    s1 = jnp.pad(jnp.eye(10, dtype=_f32), ((0, 0), (0, 22)))
    s2 = jnp.pad(jnp.eye(15, dtype=_f32), ((0, 0), (10, 7)))
    mtable, vsum = _final(acc, x1, g3_b.reshape(1, 10), s1, s2,
                          lin_w[0:10], lin_w[10:25], lin_w[25:29],
                          x2.reshape(1, 4), lin_b.reshape(1, 15),
                          lin2_w, lin2_b.reshape(1, 1))

    sdrows = _mgather(mtable, move_src, move_dst)
    wpieces = (aaa_w[0:10], aaa_w[10:20], aaa_w[20:32], aaa_w[32:46],
               aaa_w[46:47], aaa_w[47:48], aaa_b.reshape(1, 20),
               bbb_w, bbb_b.reshape(1, 1),
               ccc_w[0:10], ccc_w[10:22], ccc_w[22:23], ccc_b.reshape(1, 20),
               ddd_w, ddd_b.reshape(1, 1))
    per = _move(sdrows[0], sdrows[1], move_armies.reshape(_T, 1),
                move_type.reshape(_T, 1), wpieces)

    pparts = _pscatter(per.reshape(_T), move_ids)
    logp, v = _fin2(pparts, vsum)
    return (v.reshape(()), logp.reshape(_M))


# speedup vs baseline: 35.2882x; 1.0398x over previous
import functools

import jax
import jax.numpy as jnp
from jax import lax
from jax.experimental import pallas as pl
from jax.experimental.pallas import tpu as pltpu
from jax.experimental.pallas import tpu_sc as plsc

_N = 50000
_NP = 50048
_E = 1600000
_T = 16384
_M = 4096
_NC = 2
_NS = 16
_NW = _NC * _NS
_EPW = _E // _NW
_CH = 128
_NCH = _EPW // _CH
_TAIL = _EPW - _NCH * _CH
_RPT = _NP // _NS
_ZR = 184
_RB = 3128
_NRB = _NP // _RB
_MB = 2048
_NMB = _T // _MB
_MPW = _T // _NW

_f32 = jnp.float32
_i32 = jnp.int32

_sc_mesh = plsc.VectorSubcoreMesh(core_axis_name="c", subcore_axis_name="s")


def _tables_first_body(x1_ref, wl_ref, wr_ref, cvec_ref, xlp_ref, xrp_ref):
    x = x1_ref[...]
    xlp_ref[...] = jnp.dot(x, wl_ref[...], preferred_element_type=_f32, precision=lax.Precision.HIGHEST) + cvec_ref[...]
    xrp_ref[...] = jnp.dot(x, wr_ref[...], preferred_element_type=_f32, precision=lax.Precision.HIGHEST)


def _tables_first(x1, wlp, wrp, cvec):
    return pl.pallas_call(
        _tables_first_body,
        grid=(_NRB,),
        in_specs=[
            pl.BlockSpec((_RB, 15), lambda i: (i, 0)),
            pl.BlockSpec((15, 16), lambda i: (0, 0)),
            pl.BlockSpec((15, 16), lambda i: (0, 0)),
            pl.BlockSpec((1, 16), lambda i: (0, 0)),
        ],
        out_specs=[pl.BlockSpec((_RB, 16), lambda i: (i, 0))] * 2,
        out_shape=[jax.ShapeDtypeStruct((_NP, 16), _f32)] * 2,
    )(x1, wlp, wrp, cvec)


def _tables_next_body(acc_ref, x1_ref, b_ref, wl1_ref, wl2_ref, wr1_ref,
                      wr2_ref, cvec_ref, xlp_ref, xrp_ref):
    a = acc_ref[0] + acc_ref[1]
    x = jnp.maximum(a[:, 0:10] / (a[:, 10:11] + 1e-16) + b_ref[...], 0.0)
    x1b = x1_ref[...]
    xlp_ref[...] = (jnp.dot(x, wl1_ref[...], preferred_element_type=_f32, precision=lax.Precision.HIGHEST)
                    + jnp.dot(x1b, wl2_ref[...], preferred_element_type=_f32, precision=lax.Precision.HIGHEST)
                    + cvec_ref[...])
    xrp_ref[...] = (jnp.dot(x, wr1_ref[...], preferred_element_type=_f32, precision=lax.Precision.HIGHEST)
                    + jnp.dot(x1b, wr2_ref[...], preferred_element_type=_f32, precision=lax.Precision.HIGHEST))


def _tables_next(acc, x1, bvec, wl1, wl2, wr1, wr2, cvec):
    return pl.pallas_call(
        _tables_next_body,
        grid=(_NRB,),
        in_specs=[
            pl.BlockSpec((2, _RB, 16), lambda i: (0, i, 0)),
            pl.BlockSpec((_RB, 15), lambda i: (i, 0)),
            pl.BlockSpec((1, 10), lambda i: (0, 0)),
            pl.BlockSpec((10, 16), lambda i: (0, 0)),
            pl.BlockSpec((15, 16), lambda i: (0, 0)),
            pl.BlockSpec((10, 16), lambda i: (0, 0)),
            pl.BlockSpec((15, 16), lambda i: (0, 0)),
            pl.BlockSpec((1, 16), lambda i: (0, 0)),
        ],
        out_specs=[pl.BlockSpec((_RB, 16), lambda i: (i, 0))] * 2,
        out_shape=[jax.ShapeDtypeStruct((_NP, 16), _f32)] * 2,
    )(acc, x1, bvec, wl1, wl2, wr1, wr2, cvec)


def _final_body(acc_ref, x1_ref, b_ref, s1_ref, s2_ref, lw1_ref, lw2_ref,
                lwx2_ref, x2_ref, linb_ref, l2w_ref, l2b_ref,
                mt_ref, vs_ref):
    i = pl.program_id(0)
    a = acc_ref[0] + acc_ref[1]
    x = jnp.maximum(a[:, 0:10] / (a[:, 10:11] + 1e-16) + b_ref[...], 0.0)
    x1b = x1_ref[...]
    mt_ref[...] = (jnp.dot(x, s1_ref[...], preferred_element_type=_f32, precision=lax.Precision.HIGHEST)
                   + jnp.dot(x1b, s2_ref[...], preferred_element_type=_f32, precision=lax.Precision.HIGHEST))
    cterm = jnp.dot(x2_ref[...], lwx2_ref[...], preferred_element_type=_f32, precision=lax.Precision.HIGHEST) + linb_ref[...]
    h = jnp.maximum(jnp.dot(x, lw1_ref[...], preferred_element_type=_f32, precision=lax.Precision.HIGHEST)
                    + jnp.dot(x1b, lw2_ref[...], preferred_element_type=_f32, precision=lax.Precision.HIGHEST)
                    + cterm, 0.0)
    v = jnp.dot(h, l2w_ref[...], preferred_element_type=_f32, precision=lax.Precision.HIGHEST) + l2b_ref[...]
    rows = i * _RB + lax.broadcasted_iota(_i32, (_RB, 1), 0)
    v = jnp.where(rows < _N, v, 0.0)

    @pl.when(i == 0)
    def _():
        vs_ref[...] = jnp.zeros_like(vs_ref)

    vs_ref[...] += jnp.sum(v, keepdims=True).reshape(1, 1)


def _final(acc, x1, bvec, s1, s2, lw1, lw2, lwx2, x2r, linb, l2w, l2b):
    return pl.pallas_call(
        _final_body,
        grid=(_NRB,),
        in_specs=[
            pl.BlockSpec((2, _RB, 16), lambda i: (0, i, 0)),
            pl.BlockSpec((_RB, 15), lambda i: (i, 0)),
            pl.BlockSpec((1, 10), lambda i: (0, 0)),
            pl.BlockSpec((10, 32), lambda i: (0, 0)),
            pl.BlockSpec((15, 32), lambda i: (0, 0)),
            pl.BlockSpec((10, 15), lambda i: (0, 0)),
            pl.BlockSpec((15, 15), lambda i: (0, 0)),
            pl.BlockSpec((4, 15), lambda i: (0, 0)),
            pl.BlockSpec((1, 4), lambda i: (0, 0)),
            pl.BlockSpec((1, 15), lambda i: (0, 0)),
            pl.BlockSpec((15, 1), lambda i: (0, 0)),
            pl.BlockSpec((1, 1), lambda i: (0, 0)),
        ],
        out_specs=[
            pl.BlockSpec((_RB, 32), lambda i: (i, 0)),
            pl.BlockSpec((1, 1), lambda i: (0, 0)),
        ],
        out_shape=[
            jax.ShapeDtypeStruct((_NP, 32), _f32),
            jax.ShapeDtypeStruct((1, 1), _f32),
        ],
    )(acc, x1, bvec, s1, s2, lw1, lw2, lwx2, x2r, linb, l2w, l2b)


def _move_body(s_ref, d_ref, arm_ref, mt_ref, awxs, awxd, awx1s, awx1d,
               awarm, awext, ab, bw, bb, cwxd, cwx1d, cwarm, cb, dw, db,
               out_ref):
    s = s_ref[...]
    d = d_ref[...]
    arm = arm_ref[...]
    extra = 0.6 * arm - 0.7 * (d[:, 13:14] + d[:, 14:15])
    ha = jnp.maximum(
        jnp.dot(s[:, 0:10], awxs[...], preferred_element_type=_f32, precision=lax.Precision.HIGHEST)
        + jnp.dot(d[:, 0:10], awxd[...], preferred_element_type=_f32, precision=lax.Precision.HIGHEST)
        + jnp.dot(s[:, 13:25], awx1s[...], preferred_element_type=_f32, precision=lax.Precision.HIGHEST)
        + jnp.dot(d[:, 11:25], awx1d[...], preferred_element_type=_f32, precision=lax.Precision.HIGHEST)
        + arm * awarm[...] + extra * awext[...] + ab[...], 0.0)
    sa = jnp.dot(ha, bw[...], preferred_element_type=_f32, precision=lax.Precision.HIGHEST) + bb[...]
    hd = jnp.maximum(
        jnp.dot(d[:, 0:10], cwxd[...], preferred_element_type=_f32, precision=lax.Precision.HIGHEST)
        + jnp.dot(d[:, 13:25], cwx1d[...], preferred_element_type=_f32, precision=lax.Precision.HIGHEST)
        + arm * cwarm[...] + cb[...], 0.0)
    sd = jnp.dot(hd, dw[...], preferred_element_type=_f32, precision=lax.Precision.HIGHEST) + db[...]
    out_ref[...] = jnp.where(mt_ref[...] == 0, sa, sd)


def _move(srows, drows, arm, mtype, wpieces):
    const = lambda shape: pl.BlockSpec(shape, lambda i: (0, 0))
    return pl.pallas_call(
        _move_body,
        grid=(_NMB,),
        in_specs=[
            pl.BlockSpec((_MB, 32), lambda i: (i, 0)),
            pl.BlockSpec((_MB, 32), lambda i: (i, 0)),
            pl.BlockSpec((_MB, 1), lambda i: (i, 0)),
            pl.BlockSpec((_MB, 1), lambda i: (i, 0)),
            const((10, 20)), const((10, 20)), const((12, 20)), const((14, 20)),
            const((1, 20)), const((1, 20)), const((1, 20)),
            const((20, 1)), const((1, 1)),
            const((10, 20)), const((12, 20)), const((1, 20)), const((1, 20)),
            const((20, 1)), const((1, 1)),
        ],
        out_specs=pl.BlockSpec((_MB, 1), lambda i: (i, 0)),
        out_shape=jax.ShapeDtypeStruct((_T, 1), _f32),
    )(srows, drows, arm, mtype, *wpieces)


def _fin2_body(pp_ref, vs_ref, logp_ref, v_ref):
    a = jnp.sum(pp_ref[...], axis=0)
    p = jnp.sum(a, axis=0, keepdims=True)
    m = jnp.max(p)
    lse = m + jnp.log(jnp.sum(jnp.exp(p - m)))
    logp_ref[...] = p - lse
    v_ref[...] = jnp.tanh(vs_ref[...] / _N)


def _fin2(pparts, vsum):
    return pl.pallas_call(
        _fin2_body,
        grid=(1,),
        in_specs=[
            pl.BlockSpec((_NC, _NS, _M), lambda i: (0, 0, 0)),
            pl.BlockSpec((1, 1), lambda i: (0, 0)),
        ],
        out_specs=[
            pl.BlockSpec((1, _M), lambda i: (0, 0)),
            pl.BlockSpec((1, 1), lambda i: (0, 0)),
        ],
        out_shape=[
            jax.ShapeDtypeStruct((1, _M), _f32),
            jax.ShapeDtypeStruct((1, 1), _f32),
        ],
    )(pparts.reshape(_NC, _NS, _M), vsum)


def _edge_body(xlp, xrp, src, dst, attp, out, acc, zbuf, attv,
               idxs0, idxs1, idxs2, idxd0, idxd1, idxd2,
               sidx0, sidx1, sidx2,
               lr0, lr1, lr2, rr0, rr1, rr2, or0, or1, or2,
               idxs_t, idxd_t, lrows_t, rrows_t, orows_t,
               sbuf, exb,
               isem0, isem1, isem2, gsem0, gsem1, gsem2,
               ssem0, ssem1, ssem2, zsem, tsem):
    cid = lax.axis_index("c")
    sid = lax.axis_index("s")
    wid = cid * _NS + sid
    IDXS = (idxs0, idxs1, idxs2)
    IDXD = (idxd0, idxd1, idxd2)
    SIDX = (sidx0, sidx1, sidx2)
    LR = (lr0, lr1, lr2)
    RR = (rr0, rr1, rr2)
    OR = (or0, or1, or2)
    ISEM = (isem0, isem1, isem2)
    GSEM = (gsem0, gsem1, gsem2)
    SSEM = (ssem0, ssem1, ssem2)

    def _zb(j, c):
        zbuf[j, :] = jnp.zeros((16,), _f32)
        return c
    lax.fori_loop(0, _ZR, _zb, 0)
    pltpu.sync_copy(attp, attv)
    nz = _RPT // _ZR
    for k in range(nz):
        pltpu.make_async_copy(
            zbuf, acc.at[pl.ds(sid * _RPT + k * _ZR, _ZR)], zsem).start()
    for k in range(nz):
        pltpu.make_async_copy(
            zbuf, acc.at[pl.ds(sid * _RPT + k * _ZR, _ZR)], zsem).wait()
    plsc.subcore_barrier()

    base = wid * _EPW
    attvv = attv[...]
    ii = lax.iota(_i32, 16) * 16

    def _idx_descs(b, c):
        off = base + c * _CH
        return (pltpu.make_async_copy(src.at[pl.ds(off, _CH)], IDXS[b], ISEM[b]),
                pltpu.make_async_copy(dst.at[pl.ds(off, _CH)], IDXD[b], ISEM[b]))

    def _gath_descs(b):
        return (pltpu.make_async_copy(xlp.at[IDXS[b]], LR[b], GSEM[b]),
                pltpu.make_async_copy(xrp.at[IDXD[b]], RR[b], GSEM[b]))

    def _compute(lrows, rrows, orows):
        for g in range(_CH // 16):
            rb = g * 16
            sb = g * 256
            for j in range(16):
                t = lrows[rb + j, :] + rrows[rb + j, :]
                sbuf[pl.ds(sb + j * 16, 16)] = jnp.maximum(t, 0.2 * t) * attvv
            ev = plsc.load_gather(sbuf, [ii + sb])
            for dcol in range(1, 10):
                ev = ev + plsc.load_gather(sbuf, [ii + (sb + dcol)])
            exb[pl.ds(g * 16, 16)] = jnp.exp(ev)
        for g in range(_CH // 16):
            rb = g * 16
            for j in range(16):
                exj = plsc.load_gather(exb, [jnp.full((16,), g * 16 + j, _i32)])
                orows[rb + j, :] = exj * lrows[rb + j, :]

    def _do_chunk(k, it):
        c = it * 3 + k
        b = k
        g1, g2 = _gath_descs(b)
        g1.wait()
        g2.wait()

        @pl.when(c >= 3)
        def _():
            pltpu.make_async_copy(OR[b], acc.at[SIDX[b]], SSEM[b]).wait()
        for q in range(_CH // 16):
            SIDX[b][pl.ds(q * 16, 16)] = IDXD[b][pl.ds(q * 16, 16)]

        @pl.when(c + 3 < _NCH)
        def _():
            d1, d2 = _idx_descs(b, c + 3)
            d1.start()
            d2.start()

        @pl.when(c + 2 < _NCH)
        def _():
            b2 = (k + 2) % 3
            i1, i2 = _idx_descs(b2, c + 2)
            i1.wait()
            i2.wait()
            n1, n2 = _gath_descs(b2)
            n1.start()
            n2.start()

        _compute(LR[b], RR[b], OR[b])
        pltpu.async_copy(OR[b], acc.at[SIDX[b]], SSEM[b], add=True)

    for c0 in range(3):
        d1, d2 = _idx_descs(c0, c0)
        d1.start()
        d2.start()
    for c0 in range(2):
        d1, d2 = _idx_descs(c0, c0)
        d1.wait()
        d2.wait()
        g1, g2 = _gath_descs(c0)
        g1.start()
        g2.start()

    def _it_body(it, carry):
        _do_chunk(0, it)
        _do_chunk(1, it)
        _do_chunk(2, it)
        return carry
    lax.fori_loop(0, _NCH // 3, _it_body, 0)
    for b in range(3):
        pltpu.make_async_copy(OR[b], acc.at[SIDX[b]], SSEM[b]).wait()

    off = base + _NCH * _CH
    pltpu.sync_copy(src.at[pl.ds(off, _TAIL)], idxs_t)
    pltpu.sync_copy(dst.at[pl.ds(off, _TAIL)], idxd_t)
    pltpu.async_copy(xlp.at[idxs_t], lrows_t, tsem).wait()
    pltpu.async_copy(xrp.at[idxd_t], rrows_t, tsem).wait()

    def _tgroup(g, c):
        rb = g * 16
        for j in range(16):
            t = lrows_t[rb + j, :] + rrows_t[rb + j, :]
            sbuf[pl.ds(j * 16, 16)] = jnp.maximum(t, 0.2 * t) * attvv
        ev = plsc.load_gather(sbuf, [ii])
        for dcol in range(1, 10):
            ev = ev + plsc.load_gather(sbuf, [ii + dcol])
        exb[pl.ds(0, 16)] = jnp.exp(ev)
        for j in range(16):
            exj = plsc.load_gather(exb, [jnp.full((16,), j, _i32)])
            orows_t[rb + j, :] = exj * lrows_t[rb + j, :]
        return c
    lax.fori_loop(0, _TAIL // 16, _tgroup, 0)
    pltpu.sync_copy(orows_t, acc.at[idxd_t], add=True)

    plsc.subcore_barrier()
    pltpu.sync_copy(acc.at[pl.ds(sid * _RPT, _RPT)],
                    out.at[cid, pl.ds(sid * _RPT, _RPT)])


def _edge_pass(xlp, xrp, src, dst, attp):
    kern = pl.kernel(
        _edge_body,
        out_type=jax.ShapeDtypeStruct((_NC, _NP, 16), _f32),
        mesh=_sc_mesh,
        compiler_params=pltpu.CompilerParams(needs_layout_passes=False, use_tc_tiling_on_sc=False),
        scratch_types=[
            pltpu.VMEM_SHARED((_NP, 16), _f32),
            pltpu.VMEM((_ZR, 16), _f32),
            pltpu.VMEM((16,), _f32),
        ] + [pltpu.VMEM((_CH,), _i32)] * 9
          + [pltpu.VMEM((_CH, 16), _f32)] * 9
          + [pltpu.VMEM((_TAIL,), _i32)] * 2
          + [pltpu.VMEM((_TAIL, 16), _f32)] * 3
          + [pltpu.VMEM((2048,), _f32), pltpu.VMEM((128,), _f32)]
          + [pltpu.SemaphoreType.DMA] * 11,
    )
    return kern(xlp, xrp, src, dst, attp)


def _mgather_body(mtable, msrc, mdst, out, idx, rows, sem):
    cid = lax.axis_index("c")
    sid = lax.axis_index("s")
    base = (cid * _NS + sid) * _MPW
    for k in range(_MPW // 128):
        off = base + k * 128
        pltpu.sync_copy(msrc.at[pl.ds(off, 128)], idx)
        pltpu.async_copy(mtable.at[idx], rows, sem).wait()
        pltpu.sync_copy(rows, out.at[0, pl.ds(off, 128)])
        pltpu.sync_copy(mdst.at[pl.ds(off, 128)], idx)
        pltpu.async_copy(mtable.at[idx], rows, sem).wait()
        pltpu.sync_copy(rows, out.at[1, pl.ds(off, 128)])


def _mgather(mtable, msrc, mdst):
    kern = pl.kernel(
        _mgather_body,
        out_type=jax.ShapeDtypeStruct((2, _T, 32), _f32),
        mesh=_sc_mesh,
        compiler_params=pltpu.CompilerParams(needs_layout_passes=False, use_tc_tiling_on_sc=False),
        scratch_types=[
            pltpu.VMEM((128,), _i32),
            pltpu.VMEM((128, 32), _f32),
            pltpu.SemaphoreType.DMA,
        ],
    )
    return kern(mtable, msrc, mdst)


def _pscatter_body(per, mids, out, pbuf, ids, vals):
    cid = lax.axis_index("c")
    sid = lax.axis_index("s")
    base = (cid * _NS + sid) * _MPW

    def _z(j, c):
        pbuf[pl.ds(j * 16, 16)] = jnp.zeros((16,), _f32)
        return c
    lax.fori_loop(0, _M // 16, _z, 0)
    pltpu.sync_copy(mids.at[pl.ds(base, _MPW)], ids)
    pltpu.sync_copy(per.at[pl.ds(base, _MPW)], vals)

    def _g(j, c):
        plsc.addupdate_scatter(pbuf, [ids[pl.ds(j * 16, 16)]],
                               vals[pl.ds(j * 16, 16)])
        return c
    lax.fori_loop(0, _MPW // 16, _g, 0)
    pltpu.sync_copy(pbuf, out.at[cid, sid, 0])


def _pscatter(per, mids):
    kern = pl.kernel(
        _pscatter_body,
        out_type=jax.ShapeDtypeStruct((_NC, _NS, 1, _M), _f32),
        mesh=_sc_mesh,
        compiler_params=pltpu.CompilerParams(needs_layout_passes=False, use_tc_tiling_on_sc=False),
        scratch_types=[
            pltpu.VMEM((_M,), _f32),
            pltpu.VMEM((_MPW,), _i32),
            pltpu.VMEM((_MPW,), _f32),
        ],
    )
    return kern(per, mids)


def kernel(x1, x2, edges, move_type, move_src, move_dst, move_armies,
           move_ids, g1_wl, g1_wr, g1_att, g1_b, g2_wl, g2_wr, g2_att, g2_b,
           g3_wl, g3_wr, g3_att, g3_b, lin_w, lin_b, lin2_w, lin2_b,
           aaa_w, aaa_b, bbb_w, bbb_b, ccc_w, ccc_b, ddd_w, ddd_b):
    src = edges[0]
    dst = edges[1]
    x1 = jnp.pad(x1, ((0, _NP - _N), (0, 0)))
    cvec = jnp.zeros((1, 16), _f32).at[0, 10].set(1.0)
    pad6 = lambda w: jnp.pad(w, ((0, 0), (0, 6)))

    xlp, xrp = _tables_first(x1, pad6(g1_wl), pad6(g1_wr), cvec)
    acc = _edge_pass(xlp, xrp, src, dst, jnp.pad(g1_att, (0, 6)))
    for wl, wr, att, bprev in ((g2_wl, g2_wr, g2_att, g1_b),
                               (g3_wl, g3_wr, g3_att, g2_b)):
        xlp, xrp = _tables_next(acc, x1, bprev.reshape(1, 10),
                                pad6(wl[0:10]), pad6(wl[10:25]),
                                pad6(wr[0:10]), pad6(wr[10:25]), cvec)
        acc = _edge_pass(xlp, xrp, src, dst, jnp.pad(att, (0, 6)))

    s1 = jnp.pad(jnp.eye(10, dtype=_f32), ((0, 0), (0, 22)))
    s2 = jnp.pad(jnp.eye(15, dtype=_f32), ((0, 0), (10, 7)))
    mtable, vsum = _final(acc, x1, g3_b.reshape(1, 10), s1, s2,
                          lin_w[0:10], lin_w[10:25], lin_w[25:29],
                          x2.reshape(1, 4), lin_b.reshape(1, 15),
                          lin2_w, lin2_b.reshape(1, 1))

    sdrows = _mgather(mtable, move_src, move_dst)
    wpieces = (aaa_w[0:10], aaa_w[10:20], aaa_w[20:32], aaa_w[32:46],
               aaa_w[46:47], aaa_w[47:48], aaa_b.reshape(1, 20),
               bbb_w, bbb_b.reshape(1, 1),
               ccc_w[0:10], ccc_w[10:22], ccc_w[22:23], ccc_b.reshape(1, 20),
               ddd_w, ddd_b.reshape(1, 1))
    per = _move(sdrows[0], sdrows[1], move_armies.reshape(_T, 1),
                move_type.reshape(_T, 1), wpieces)

    pparts = _pscatter(per.reshape(_T), move_ids)
    logp, v = _fin2(pparts, vsum)
    return (v.reshape(()), logp.reshape(_M))
